# Initial kernel scaffold; baseline (speedup 1.0000x reference)
import jax
import jax.numpy as jnp
from jax.experimental import pallas as pl


def kernel(indices, table):
    raise NotImplementedError("write your pallas kernel here")


# speedup vs baseline: 1.6849x; 1.6849x over previous
import functools

import jax
import jax.numpy as jnp
from jax import lax
from jax.experimental import pallas as pl
from jax.experimental.pallas import tpu as pltpu
from jax.experimental.pallas import tpu_sc as plsc

_D = 64
_B = 16384 * 50
_NC = 2
_NS = 16
_NW = _NC * _NS
_BPW = _B // _NW
_K = 128
_CHUNKS = _BPW // _K

_mesh = plsc.VectorSubcoreMesh(core_axis_name="c", subcore_axis_name="s")


@functools.partial(
    pl.kernel,
    mesh=_mesh,
    out_type=jax.ShapeDtypeStruct((_B, _D), jnp.float32),
    scratch_types=[
        pltpu.VMEM((_BPW,), jnp.int32),
        pltpu.VMEM((_K, _D), jnp.float32),
        pltpu.SemaphoreType.DMA,
    ],
    compiler_params=pltpu.CompilerParams(use_tc_tiling_on_sc=False),
)
def _gather_kernel(idx_hbm, table_hbm, out_hbm, idx_v, rows_v, sem):
    wid = lax.axis_index("s") * _NC + lax.axis_index("c")
    base = wid * _BPW
    pltpu.sync_copy(idx_hbm.at[pl.ds(base, _BPW)], idx_v)

    def body(g, carry):
        off = g * _K
        pltpu.async_copy(
            table_hbm.at[idx_v.at[pl.ds(off, _K)]], rows_v, sem
        ).wait()
        pltpu.sync_copy(rows_v, out_hbm.at[pl.ds(base + off, _K)])
        return carry

    lax.fori_loop(0, _CHUNKS, body, 0)


def kernel(indices, table):
    idx_flat = indices.reshape(-1)
    out = _gather_kernel(idx_flat, table)
    return out.reshape(indices.shape[0], indices.shape[1], _D)


# speedup vs baseline: 1.8720x; 1.1110x over previous
import functools

import jax
import jax.numpy as jnp
from jax import lax
from jax.experimental import pallas as pl
from jax.experimental.pallas import tpu as pltpu
from jax.experimental.pallas import tpu_sc as plsc

_D = 64
_B = 16384 * 50
_NC = 2
_NS = 16
_NW = _NC * _NS
_BPW = _B // _NW
_K = 128
_CHUNKS = _BPW // _K
_S = 8
_NPASS = _CHUNKS // _S

_mesh = plsc.VectorSubcoreMesh(core_axis_name="c", subcore_axis_name="s")


@functools.partial(
    pl.kernel,
    mesh=_mesh,
    out_type=jax.ShapeDtypeStruct((_B, _D), jnp.float32),
    scratch_types=[
        pltpu.VMEM((_BPW,), jnp.int32),
        pltpu.VMEM((_S, _K, _D), jnp.float32),
        pltpu.SemaphoreType.DMA((_S,)),
        pltpu.SemaphoreType.DMA((_S,)),
    ],
    compiler_params=pltpu.CompilerParams(use_tc_tiling_on_sc=False),
)
def _gather_kernel(idx_hbm, table_hbm, out_hbm, idx_v, rows_v, sem_g, sem_o):
    wid = lax.axis_index("s") * _NC + lax.axis_index("c")
    base = wid * _BPW
    pltpu.sync_copy(idx_hbm.at[pl.ds(base, _BPW)], idx_v)

    for b in range(_S):
        pltpu.async_copy(
            table_hbm.at[idx_v.at[pl.ds(b * _K, _K)]],
            rows_v.at[b],
            sem_g.at[b],
        )

    def ring_pass(p, carry):
        off0 = p * _S * _K
        for b in range(_S):
            pltpu.make_async_copy(
                table_hbm.at[idx_v.at[pl.ds(b * _K, _K)]],
                rows_v.at[b],
                sem_g.at[b],
            ).wait()
            pltpu.async_copy(
                rows_v.at[b],
                out_hbm.at[pl.ds(base + off0 + b * _K, _K)],
                sem_o.at[b],
            )
        for b in range(_S):
            pltpu.make_async_copy(
                rows_v.at[b],
                out_hbm.at[pl.ds(base + off0 + b * _K, _K)],
                sem_o.at[b],
            ).wait()

            @pl.when(p + 1 < _NPASS)
            def _():
                pltpu.async_copy(
                    table_hbm.at[idx_v.at[pl.ds(off0 + (_S + b) * _K, _K)]],
                    rows_v.at[b],
                    sem_g.at[b],
                )

        return carry

    lax.fori_loop(0, _NPASS, ring_pass, 0)


def kernel(indices, table):
    idx_flat = indices.reshape(-1)
    out = _gather_kernel(idx_flat, table)
    return out.reshape(indices.shape[0], indices.shape[1], _D)


# speedup vs baseline: 1.8786x; 1.0035x over previous
import functools

import jax
import jax.numpy as jnp
from jax import lax
from jax.experimental import pallas as pl
from jax.experimental.pallas import tpu as pltpu
from jax.experimental.pallas import tpu_sc as plsc

_D = 64
_BATCH = 16384
_L = 50
_NC = 2
_NS = 16
_NW = _NC * _NS
_SPW = _BATCH // _NW
_S = 8
_NPASS = _SPW // _S

_mesh = plsc.VectorSubcoreMesh(core_axis_name="c", subcore_axis_name="s")


@functools.partial(
    pl.kernel,
    mesh=_mesh,
    out_type=jax.ShapeDtypeStruct((_BATCH, _L, _D), jnp.float32),
    scratch_types=[
        pltpu.VMEM((_SPW, _L), jnp.int32),
        pltpu.VMEM((_S, _L, _D), jnp.float32),
        pltpu.SemaphoreType.DMA((_S,)),
        pltpu.SemaphoreType.DMA((_S,)),
    ],
    compiler_params=pltpu.CompilerParams(use_tc_tiling_on_sc=False),
)
def _gather_kernel(idx_hbm, table_hbm, out_hbm, idx_v, rows_v, sem_g, sem_o):
    wid = lax.axis_index("s") * _NC + lax.axis_index("c")
    base = wid * _SPW
    pltpu.sync_copy(idx_hbm.at[pl.ds(base, _SPW)], idx_v)

    for b in range(_S):
        pltpu.async_copy(
            table_hbm.at[idx_v.at[b]],
            rows_v.at[b],
            sem_g.at[b],
        )

    def ring_pass(p, carry):
        seq0 = p * _S
        for b in range(_S):
            pltpu.make_async_copy(
                table_hbm.at[idx_v.at[b]],
                rows_v.at[b],
                sem_g.at[b],
            ).wait()
            pltpu.async_copy(
                rows_v.at[b],
                out_hbm.at[base + seq0 + b],
                sem_o.at[b],
            )
        for b in range(_S):
            pltpu.make_async_copy(
                rows_v.at[b],
                out_hbm.at[base + seq0 + b],
                sem_o.at[b],
            ).wait()

            @pl.when(p + 1 < _NPASS)
            def _():
                pltpu.async_copy(
                    table_hbm.at[idx_v.at[seq0 + _S + b]],
                    rows_v.at[b],
                    sem_g.at[b],
                )

        return carry

    lax.fori_loop(0, _NPASS, ring_pass, 0)


def kernel(indices, table):
    return _gather_kernel(indices, table)
